# Initial kernel scaffold; baseline (speedup 1.0000x reference)
import jax
import jax.numpy as jnp
from jax.experimental import pallas as pl


def kernel(speaker, turn, speaker_table, turn_table, position_table):
    raise NotImplementedError("write your pallas kernel here")


# speedup vs baseline: 6.1557x; 6.1557x over previous
import jax
import jax.numpy as jnp
from jax.experimental import pallas as pl


def _path_kernel(s_ref, t_ref, tab_ref, o_ref, *, chunk, n):
    k = pl.program_id(1)
    tp = tab_ref.shape[0]

    s = s_ref[0, 0]
    t = t_ref[0, 0]

    r = jax.lax.broadcasted_iota(jnp.int32, (1, chunk), 1) + k * chunk
    i = r // n
    j = r - i * n
    d = jnp.clip(j - i, -17, 17) + 17

    c = jax.lax.broadcasted_iota(jnp.int32, (tp, chunk), 0)
    onehot_t = ((c == s) | (c == t + 2) | (c == d + 4)).astype(jnp.float32)

    o_ref[0] = jax.lax.dot_general(
        onehot_t, tab_ref[...],
        dimension_numbers=(((0,), (0,)), ((), ())),
        preferred_element_type=jnp.float32,
    )


def kernel(speaker, turn, speaker_table, turn_table, position_table):
    b, n, _ = speaker.shape
    hq = speaker_table.shape[1]
    p = position_table.shape[0]
    h = 2 * hq + position_table.shape[1]
    f = n * n

    tp = ((4 + p + 7) // 8) * 8
    table = jnp.zeros((tp, h), jnp.float32)
    table = table.at[0:2, 0:hq].set(speaker_table)
    table = table.at[2:4, hq:2 * hq].set(turn_table)
    table = table.at[4:4 + p, 2 * hq:].set(position_table)

    chunk = 2000
    kk = f // chunk

    s4 = speaker.reshape(b, kk, 1, chunk).astype(jnp.int32)
    t4 = turn.reshape(b, kk, 1, chunk).astype(jnp.int32)

    out = pl.pallas_call(
        lambda s_ref, t_ref, tab_ref, o_ref: _path_kernel(
            s_ref, t_ref, tab_ref, o_ref, chunk=chunk, n=n),
        grid=(b, kk),
        in_specs=[
            pl.BlockSpec((1, 1, 1, chunk), lambda bi, ki: (bi, ki, 0, 0)),
            pl.BlockSpec((1, 1, 1, chunk), lambda bi, ki: (bi, ki, 0, 0)),
            pl.BlockSpec((tp, h), lambda bi, ki: (0, 0)),
        ],
        out_specs=pl.BlockSpec((1, chunk, h), lambda bi, ki: (bi, ki, 0)),
        out_shape=jax.ShapeDtypeStruct((b, f, h), jnp.float32),
    )(s4, t4, table)

    return out.reshape(b, n, n, h)
